# baseline (speedup 1.0000x reference)
import functools

import jax
import jax.numpy as jnp
from jax import lax
from jax.experimental import pallas as pl
from jax.experimental.pallas import tpu as pltpu
from jax.experimental.pallas import tpu_sc as plsc

B, H, W = 16, 512, 512
NW = 32
ROWS_PER_W = (B * H) // NW
RW3 = W * 3


def _body(xf, out, row_v, idx_v, gth_v, wgt_v, out_v, sem):
    wid = lax.axis_index("c") * 16 + lax.axis_index("s")
    b = wid // 2
    row0 = (wid % 2) * ROWS_PER_W + b * H

    lanes = lax.iota(jnp.int32, 16)

    def do_row(r, _):
        rg = row0 + r
        yrow = rg - b * H
        pltpu.sync_copy(xf.at[pl.ds(rg * RW3, RW3)], row_v)
        yrow_f = yrow.astype(jnp.float32)

        def gen(g):
            j = g * 16 + lanes
            j3 = j * 3
            wx = plsc.load_gather(row_v, [j3 + 1])
            wy = plsc.load_gather(row_v, [j3 + 2])
            fx = j.astype(jnp.float32) + wx
            fy = yrow_f + wy
            cx = jnp.clip(fx.astype(jnp.int32), 0, W - 2)
            cy = jnp.clip(fy.astype(jnp.int32), 0, H - 2)
            dx = fx - cx.astype(jnp.float32)
            dy = fy - cy.astype(jnp.float32)
            base3 = ((b * H + cy) * W + cx) * 3
            o = g * 16
            idx_v[pl.ds(o, 16)] = base3
            idx_v[pl.ds(W + o, 16)] = base3 + 3
            idx_v[pl.ds(2 * W + o, 16)] = base3 + RW3
            idx_v[pl.ds(3 * W + o, 16)] = base3 + RW3 + 3
            wgt_v[pl.ds(o, 16)] = dx
            wgt_v[pl.ds(W + o, 16)] = dy

        for g in range(W // 16):
            gen(g)
        pltpu.async_copy(xf.at[idx_v], gth_v, sem).wait()

        def comb(g):
            o = g * 16
            g00 = gth_v[pl.ds(o, 16)]
            g01 = gth_v[pl.ds(W + o, 16)]
            g10 = gth_v[pl.ds(2 * W + o, 16)]
            g11 = gth_v[pl.ds(3 * W + o, 16)]
            dx = wgt_v[pl.ds(o, 16)]
            dy = wgt_v[pl.ds(W + o, 16)]
            ndx = 1.0 - dx
            ndy = 1.0 - dy
            out_v[pl.ds(o, 16)] = (g00 * ndx * ndy + g01 * dx * ndy
                                   + g11 * dx * dy + g10 * ndx * dy)

        for g in range(W // 16):
            comb(g)
        pltpu.sync_copy(out_v, out.at[pl.ds(rg * W, W)])
        return 0

    lax.fori_loop(0, ROWS_PER_W, do_row, 0)


@jax.jit
def kernel(x):
    xf = x.reshape(B * H * W * 3)
    mesh = plsc.VectorSubcoreMesh(core_axis_name="c", subcore_axis_name="s")
    call = pl.kernel(
        _body,
        out_type=jax.ShapeDtypeStruct((B * H * W,), jnp.float32),
        mesh=mesh,
        scratch_types=[
            pltpu.VMEM((RW3,), jnp.float32),
            pltpu.VMEM((4 * W,), jnp.int32),
            pltpu.VMEM((4 * W,), jnp.float32),
            pltpu.VMEM((2 * W,), jnp.float32),
            pltpu.VMEM((W,), jnp.float32),
            pltpu.SemaphoreType.DMA,
        ],
        compiler_params=pltpu.CompilerParams(
            needs_layout_passes=False, use_tc_tiling_on_sc=True),
    )
    y = call(xf)
    return y.reshape(B, H, W, 1)


# speedup vs baseline: 5.4408x; 5.4408x over previous
import functools

import jax
import jax.numpy as jnp
from jax import lax
from jax.experimental import pallas as pl
from jax.experimental.pallas import tpu as pltpu
from jax.experimental.pallas import tpu_sc as plsc

B, H, W = 16, 512, 512
NW = 32
ROWS_PER_W = (B * H) // NW
NPIX = B * H * W


def _reformat_body(x_ref, inp_ref, wx_ref, wy_ref):
    xb = x_ref[0]
    inp_ref[0] = xb[0].reshape(H * W // 128, 128)
    wx_ref[0] = xb[1].reshape(H * W // 128, 128)
    wy_ref[0] = xb[2].reshape(H * W // 128, 128)


def _warp_body(inp, wxp, wyp, out, row_v, idx_v, gth_v, wgt_v, out_v, sem):
    wid = lax.axis_index("c") * 16 + lax.axis_index("s")
    b = wid // 2
    row0 = (wid % 2) * ROWS_PER_W + b * H

    lanes = lax.iota(jnp.int32, 16)

    def do_row(r, _):
        rg = row0 + r
        yrow = rg - b * H
        pltpu.sync_copy(wxp.at[pl.ds(rg * W, W)], row_v.at[pl.ds(0, W)])
        pltpu.sync_copy(wyp.at[pl.ds(rg * W, W)], row_v.at[pl.ds(W, W)])
        yrow_f = yrow.astype(jnp.float32)

        def gen(g):
            j = g * 16 + lanes
            wx = row_v[pl.ds(g * 16, 16)]
            wy = row_v[pl.ds(W + g * 16, 16)]
            fx = j.astype(jnp.float32) + wx
            fy = yrow_f + wy
            cx = jnp.clip(fx.astype(jnp.int32), 0, W - 2)
            cy = jnp.clip(fy.astype(jnp.int32), 0, H - 2)
            dx = fx - cx.astype(jnp.float32)
            dy = fy - cy.astype(jnp.float32)
            base = (b * H + cy) * W + cx
            o = g * 16
            idx_v[pl.ds(o, 16)] = base
            idx_v[pl.ds(W + o, 16)] = base + 1
            idx_v[pl.ds(2 * W + o, 16)] = base + W
            idx_v[pl.ds(3 * W + o, 16)] = base + W + 1
            wgt_v[pl.ds(o, 16)] = dx
            wgt_v[pl.ds(W + o, 16)] = dy

        for g in range(W // 16):
            gen(g)
        pltpu.async_copy(inp.at[idx_v], gth_v, sem).wait()

        def comb(g):
            o = g * 16
            g00 = gth_v[pl.ds(o, 16)]
            g01 = gth_v[pl.ds(W + o, 16)]
            g10 = gth_v[pl.ds(2 * W + o, 16)]
            g11 = gth_v[pl.ds(3 * W + o, 16)]
            dx = wgt_v[pl.ds(o, 16)]
            dy = wgt_v[pl.ds(W + o, 16)]
            ndx = 1.0 - dx
            ndy = 1.0 - dy
            out_v[pl.ds(o, 16)] = (g00 * ndx * ndy + g01 * dx * ndy
                                   + g11 * dx * dy + g10 * ndx * dy)

        for g in range(W // 16):
            comb(g)
        pltpu.sync_copy(out_v, out.at[pl.ds(rg * W, W)])
        return 0

    lax.fori_loop(0, ROWS_PER_W, do_row, 0)


@jax.jit
def kernel(x):
    xt = jnp.transpose(x, (0, 3, 1, 2))
    plane = jax.ShapeDtypeStruct((B, H * W // 128, 128), jnp.float32)
    inp3, wx3, wy3 = pl.pallas_call(
        _reformat_body,
        grid=(B,),
        in_specs=[pl.BlockSpec((1, 3, H, W), lambda b: (b, 0, 0, 0))],
        out_specs=[pl.BlockSpec((1, H * W // 128, 128), lambda b: (b, 0, 0))] * 3,
        out_shape=[plane] * 3,
    )(xt)
    inp1 = inp3.reshape(NPIX)
    wx1 = wx3.reshape(NPIX)
    wy1 = wy3.reshape(NPIX)

    mesh = plsc.VectorSubcoreMesh(core_axis_name="c", subcore_axis_name="s")
    call = pl.kernel(
        _warp_body,
        out_type=jax.ShapeDtypeStruct((NPIX,), jnp.float32),
        mesh=mesh,
        scratch_types=[
            pltpu.VMEM((2 * W,), jnp.float32),
            pltpu.VMEM((4 * W,), jnp.int32),
            pltpu.VMEM((4 * W,), jnp.float32),
            pltpu.VMEM((2 * W,), jnp.float32),
            pltpu.VMEM((W,), jnp.float32),
            pltpu.SemaphoreType.DMA,
        ],
        compiler_params=pltpu.CompilerParams(needs_layout_passes=False),
    )
    y = call(inp1, wx1, wy1)
    return y.reshape(B, H, W, 1)


# speedup vs baseline: 6.8615x; 1.2611x over previous
import functools

import jax
import jax.numpy as jnp
from jax import lax
from jax.experimental import pallas as pl
from jax.experimental.pallas import tpu as pltpu
from jax.experimental.pallas import tpu_sc as plsc

B, H, W = 16, 512, 512
NW = 32
ROWS_PER_W = (B * H) // NW
NPIX = B * H * W
K = W // 4
CH = 8


def _reformat_body(x_ref, inp_ref, wx_ref, wy_ref):
    xb = x_ref[0]
    inp_ref[0] = xb[0].reshape(H * W // 128, 128)
    wx_ref[0] = xb[1].reshape(H * W // 128, 128)
    wy_ref[0] = xb[2].reshape(H * W // 128, 128)


def _build_body(inp, tab, rows_v, trow_v, sem):
    wid = lax.axis_index("c") * 16 + lax.axis_index("s")
    b = wid // 2
    y0 = (wid % 2) * ROWS_PER_W
    lanes = lax.iota(jnp.int32, 16)

    def chunk(ci, _):
        gy = b * H + y0 + ci * CH
        pltpu.sync_copy(inp.at[pl.ds(gy * W, CH * W)], rows_v.at[pl.ds(0, CH * W)])
        last = jnp.minimum(gy + CH, B * H - 1)
        pltpu.sync_copy(inp.at[pl.ds(last * W, W)], rows_v.at[pl.ds(CH * W, W)])
        def row_body(y2, _):
            base = y2 * (16 * K)
            for g in range(W // 16):
                c = g * 16 + lanes
                k16 = (c >> 2) << 4
                m0 = c & 3
                i_hi = base + k16 + m0
                i_lo = base + jnp.maximum(k16 - 16, 0) + m0 + 4
                msk = c >= 4
                vy = rows_v[pl.ds(y2 * W + g * 16, 16)]
                vy1 = rows_v[pl.ds((y2 + 1) * W + g * 16, 16)]
                plsc.store_scatter(trow_v, [i_hi], vy)
                plsc.store_scatter(trow_v, [i_lo], vy, mask=msk)
                plsc.store_scatter(trow_v, [i_hi + 8], vy1)
                plsc.store_scatter(trow_v, [i_lo + 8], vy1, mask=msk)
            return 0

        lax.fori_loop(0, CH, row_body, 0)
        pltpu.sync_copy(trow_v, tab.at[pl.ds(gy * (16 * K), CH * 16 * K)])
        return 0

    lax.fori_loop(0, ROWS_PER_W // CH, chunk, 0)


def _warp_body(tab, wxp, wyp, out, row_v, idx_v, off_v, gth_v, wgt_v, out_v,
               sem):
    wid = lax.axis_index("c") * 16 + lax.axis_index("s")
    b = wid // 2
    row0 = (wid % 2) * ROWS_PER_W + b * H

    lanes = lax.iota(jnp.int32, 16)

    def do_row(r, _):
        rg = row0 + r
        yrow = rg - b * H
        pltpu.sync_copy(wxp.at[pl.ds(rg * W, W)], row_v.at[pl.ds(0, W)])
        pltpu.sync_copy(wyp.at[pl.ds(rg * W, W)], row_v.at[pl.ds(W, W)])
        yrow_f = yrow.astype(jnp.float32)

        def gen(g):
            j = g * 16 + lanes
            wx = row_v[pl.ds(g * 16, 16)]
            wy = row_v[pl.ds(W + g * 16, 16)]
            fx = j.astype(jnp.float32) + wx
            fy = yrow_f + wy
            cx = jnp.clip(fx.astype(jnp.int32), 0, W - 2)
            cy = jnp.clip(fy.astype(jnp.int32), 0, H - 2)
            dx = fx - cx.astype(jnp.float32)
            dy = fy - cy.astype(jnp.float32)
            o = g * 16
            idx_v[pl.ds(o, 16)] = (b * H + cy) * K + (cx >> 2)
            off_v[pl.ds(o, 16)] = cx & 3
            wgt_v[pl.ds(o, 16)] = dx
            wgt_v[pl.ds(W + o, 16)] = dy

        for g in range(W // 16):
            gen(g)
        pltpu.async_copy(tab.at[idx_v], gth_v, sem).wait()

        def comb(g):
            o = g * 16
            prow = o + lanes
            off = off_v[pl.ds(o, 16)]
            g00 = plsc.load_gather(gth_v, [prow, off])
            g01 = plsc.load_gather(gth_v, [prow, off + 1])
            g10 = plsc.load_gather(gth_v, [prow, off + 8])
            g11 = plsc.load_gather(gth_v, [prow, off + 9])
            dx = wgt_v[pl.ds(o, 16)]
            dy = wgt_v[pl.ds(W + o, 16)]
            ndx = 1.0 - dx
            ndy = 1.0 - dy
            out_v[pl.ds(o, 16)] = (g00 * ndx * ndy + g01 * dx * ndy
                                   + g11 * dx * dy + g10 * ndx * dy)

        for g in range(W // 16):
            comb(g)
        pltpu.sync_copy(out_v, out.at[pl.ds(rg * W, W)])
        return 0

    lax.fori_loop(0, ROWS_PER_W, do_row, 0)


@jax.jit
def kernel(x):
    xt = jnp.transpose(x, (0, 3, 1, 2))
    plane = jax.ShapeDtypeStruct((B, H * W // 128, 128), jnp.float32)
    inp3, wx3, wy3 = pl.pallas_call(
        _reformat_body,
        grid=(B,),
        in_specs=[pl.BlockSpec((1, 3, H, W), lambda b: (b, 0, 0, 0))],
        out_specs=[pl.BlockSpec((1, H * W // 128, 128), lambda b: (b, 0, 0))] * 3,
        out_shape=[plane] * 3,
    )(xt)
    inp1 = inp3.reshape(NPIX)
    wx1 = wx3.reshape(NPIX)
    wy1 = wy3.reshape(NPIX)

    mesh = plsc.VectorSubcoreMesh(core_axis_name="c", subcore_axis_name="s")
    sc_params = pltpu.CompilerParams(
        needs_layout_passes=False, use_tc_tiling_on_sc=False)

    build = pl.kernel(
        _build_body,
        out_type=jax.ShapeDtypeStruct((4 * NPIX,), jnp.float32),
        mesh=mesh,
        scratch_types=[
            pltpu.VMEM(((CH + 1) * W,), jnp.float32),
            pltpu.VMEM((CH * 16 * K,), jnp.float32),
            pltpu.SemaphoreType.DMA,
        ],
        compiler_params=sc_params,
    )
    tab2 = build(inp1).reshape(B * H * K, 16)

    warp = pl.kernel(
        _warp_body,
        out_type=jax.ShapeDtypeStruct((NPIX,), jnp.float32),
        mesh=mesh,
        scratch_types=[
            pltpu.VMEM((2 * W,), jnp.float32),
            pltpu.VMEM((W,), jnp.int32),
            pltpu.VMEM((W,), jnp.int32),
            pltpu.VMEM((W, 16), jnp.float32),
            pltpu.VMEM((2 * W,), jnp.float32),
            pltpu.VMEM((W,), jnp.float32),
            pltpu.SemaphoreType.DMA,
        ],
        compiler_params=sc_params,
    )
    y = warp(tab2, wx1, wy1)
    return y.reshape(B, H, W, 1)


# speedup vs baseline: 10.0275x; 1.4614x over previous
import functools

import jax
import jax.numpy as jnp
from jax import lax
from jax.experimental import pallas as pl
from jax.experimental.pallas import tpu as pltpu
from jax.experimental.pallas import tpu_sc as plsc

B, H, W = 16, 512, 512
NW = 32
ROWS_PER_W = (B * H) // NW
NPIX = B * H * W
K = W // 4
CH = 8


def _reformat_body(x_ref, inp_ref, wxy_ref):
    xb = x_ref[0]
    inp_ref[0] = xb[0].reshape(H * W // 128, 128)
    wxy = jnp.stack([xb[1], xb[2]], axis=1)
    wxy_ref[0] = wxy.reshape(2 * H * W // 128, 128)


def _fused_body(inp, wxy, out, tab, rows_v, trow_v, wrow_v, idx0_v, idx1_v,
                off0_v, off1_v, gth0_v, gth1_v, wgt0_v, wgt1_v, out_v,
                bsem, gsem):
    wid = lax.axis_index("c") * 16 + lax.axis_index("s")
    b = wid // 2
    y0 = (wid % 2) * ROWS_PER_W
    lanes = lax.iota(jnp.int32, 16)

    def chunk(ci, _):
        gy = b * H + y0 + ci * CH
        pltpu.sync_copy(inp.at[pl.ds(gy * W, CH * W)],
                        rows_v.at[pl.ds(0, CH * W)])
        last = jnp.minimum(gy + CH, B * H - 1)
        pltpu.sync_copy(inp.at[pl.ds(last * W, W)],
                        rows_v.at[pl.ds(CH * W, W)])

        def row_body(y2, _):
            for g in range(W // 16):
                c = g * 16 + lanes
                k = c >> 2
                m0 = c & 3
                r_hi = y2 * K + k
                r_lo = y2 * K + jnp.maximum(k - 1, 0)
                msk = c >= 4
                vy = rows_v[pl.ds(y2 * W + g * 16, 16)]
                vy1 = rows_v[pl.ds((y2 + 1) * W + g * 16, 16)]
                plsc.store_scatter(trow_v, [r_hi, m0], vy)
                plsc.store_scatter(trow_v, [r_lo, m0 + 4], vy, mask=msk)
                plsc.store_scatter(trow_v, [r_hi, m0 + 8], vy1)
                plsc.store_scatter(trow_v, [r_lo, m0 + 12], vy1, mask=msk)
            return 0

        lax.fori_loop(0, CH, row_body, 0)
        pltpu.sync_copy(trow_v, tab.at[pl.ds(gy * K, CH * K)])
        return 0

    lax.fori_loop(0, ROWS_PER_W // CH, chunk, 0)
    plsc.subcore_barrier()

    row0 = (wid % 2) * ROWS_PER_W + b * H

    bufs = ((idx0_v, off0_v, gth0_v, wgt0_v), (idx1_v, off1_v, gth1_v, wgt1_v))

    def gen(rg, yrow, par):
        idx_v, off_v, _, wgt_v = bufs[par]
        pltpu.sync_copy(wxy.at[pl.ds(rg * 2 * W, 2 * W)], wrow_v)
        yrow_f = yrow.astype(jnp.float32)
        for g in range(W // 16):
            j = g * 16 + lanes
            wx = wrow_v[pl.ds(g * 16, 16)]
            wy = wrow_v[pl.ds(W + g * 16, 16)]
            fx = j.astype(jnp.float32) + wx
            fy = yrow_f + wy
            cx = jnp.clip(fx.astype(jnp.int32), 0, W - 2)
            cy = jnp.clip(fy.astype(jnp.int32), 0, H - 2)
            dx = fx - cx.astype(jnp.float32)
            dy = fy - cy.astype(jnp.float32)
            o = g * 16
            idx_v[pl.ds(o, 16)] = (b * H + cy) * K + (cx >> 2)
            off_v[pl.ds(o, 16)] = cx & 3
            wgt_v[pl.ds(o, 16)] = dx
            wgt_v[pl.ds(W + o, 16)] = dy

    def fire(par):
        idx_v, _, gth_v, _ = bufs[par]
        return pltpu.async_copy(tab.at[idx_v], gth_v, gsem.at[par])

    def comb_row(rg, par):
        idx_v, off_v, gth_v, wgt_v = bufs[par]
        pltpu.make_async_copy(tab.at[idx_v], gth_v, gsem.at[par]).wait()
        for g in range(W // 16):
            o = g * 16
            prow = o + lanes
            off = off_v[pl.ds(o, 16)]
            g00 = plsc.load_gather(gth_v, [prow, off])
            g01 = plsc.load_gather(gth_v, [prow, off + 1])
            g10 = plsc.load_gather(gth_v, [prow, off + 8])
            g11 = plsc.load_gather(gth_v, [prow, off + 9])
            dx = wgt_v[pl.ds(o, 16)]
            dy = wgt_v[pl.ds(W + o, 16)]
            ndx = 1.0 - dx
            ndy = 1.0 - dy
            out_v[pl.ds(o, 16)] = (g00 * ndx * ndy + g01 * dx * ndy
                                   + g11 * dx * dy + g10 * ndx * dy)
        pltpu.sync_copy(out_v, out.at[pl.ds(rg * W, W)])

    gen(row0, y0, 0)
    fire(0)

    def do_pair(rr, _):
        r1 = 2 * rr + 1
        gen(row0 + r1, y0 + r1, 1)
        fire(1)
        comb_row(row0 + r1 - 1, 0)
        r2 = r1 + 1
        gen(row0 + r2, y0 + r2, 0)
        fire(0)
        comb_row(row0 + r2 - 1, 1)
        return 0

    lax.fori_loop(0, (ROWS_PER_W - 2) // 2, do_pair, 0)
    rl = ROWS_PER_W - 1
    gen(row0 + rl, y0 + rl, 1)
    fire(1)
    comb_row(row0 + rl - 1, 0)
    comb_row(row0 + rl, 1)


@jax.jit
def kernel(x):
    xt = jnp.transpose(x, (0, 3, 1, 2))
    plane = jax.ShapeDtypeStruct((B, H * W // 128, 128), jnp.float32)
    wplane = jax.ShapeDtypeStruct((B, 2 * H * W // 128, 128), jnp.float32)
    inp3, wxy3 = pl.pallas_call(
        _reformat_body,
        grid=(B,),
        in_specs=[pl.BlockSpec((1, 3, H, W), lambda b: (b, 0, 0, 0))],
        out_specs=[
            pl.BlockSpec((1, H * W // 128, 128), lambda b: (b, 0, 0)),
            pl.BlockSpec((1, 2 * H * W // 128, 128), lambda b: (b, 0, 0)),
        ],
        out_shape=[plane, wplane],
    )(xt)
    inp1 = inp3.reshape(NPIX)
    wxy1 = wxy3.reshape(2 * NPIX)

    mesh = plsc.VectorSubcoreMesh(core_axis_name="c", subcore_axis_name="s")
    sc_params = pltpu.CompilerParams(
        needs_layout_passes=False, use_tc_tiling_on_sc=False)

    fused = pl.kernel(
        _fused_body,
        out_type=[
            jax.ShapeDtypeStruct((NPIX,), jnp.float32),
            jax.ShapeDtypeStruct((B * H * K, 16), jnp.float32),
        ],
        mesh=mesh,
        scratch_types=[
            pltpu.VMEM(((CH + 1) * W,), jnp.float32),
            pltpu.VMEM((CH * K, 16), jnp.float32),
            pltpu.VMEM((2 * W,), jnp.float32),
            pltpu.VMEM((W,), jnp.int32),
            pltpu.VMEM((W,), jnp.int32),
            pltpu.VMEM((W,), jnp.int32),
            pltpu.VMEM((W,), jnp.int32),
            pltpu.VMEM((W, 16), jnp.float32),
            pltpu.VMEM((W, 16), jnp.float32),
            pltpu.VMEM((2 * W,), jnp.float32),
            pltpu.VMEM((2 * W,), jnp.float32),
            pltpu.VMEM((W,), jnp.float32),
            pltpu.SemaphoreType.DMA,
            pltpu.SemaphoreType.DMA((2,)),
        ],
        compiler_params=sc_params,
    )
    y, _ = fused(inp1, wxy1)
    return y.reshape(B, H, W, 1)


# speedup vs baseline: 12.2065x; 1.2173x over previous
import functools

import jax
import jax.numpy as jnp
from jax import lax
from jax.experimental import pallas as pl
from jax.experimental.pallas import tpu as pltpu
from jax.experimental.pallas import tpu_sc as plsc

B, H, W = 16, 512, 512
NW = 32
ROWS_PER_W = (B * H) // NW
NPIX = B * H * W
K = W // 4
CH = 8


def _reformat_body(x_ref, inp_ref, wxy_ref):
    xb = x_ref[0]
    inp_ref[0] = xb[0].reshape(H * W // 128, 128)
    wxy = jnp.stack([xb[1], xb[2]], axis=1)
    wxy_ref[0] = wxy.reshape(2 * H * W // 128, 128)


def _fused_body(inp, wxy, out, tab, rows_v, trow_v, wrow0_v, wrow1_v,
                idx0_v, idx1_v, off0_v, off1_v, gth0_v, gth1_v,
                wgt0_v, wgt1_v, out_v, wsem, gsem):
    wid = lax.axis_index("c") * 16 + lax.axis_index("s")
    b = wid // 2
    y0 = (wid % 2) * ROWS_PER_W
    lanes = lax.iota(jnp.int32, 16)

    def chunk(ci, _):
        gy = b * H + y0 + ci * CH
        pltpu.sync_copy(inp.at[pl.ds(gy * W, CH * W)],
                        rows_v.at[pl.ds(0, CH * W)])
        last = jnp.minimum(gy + CH, B * H - 1)
        pltpu.sync_copy(inp.at[pl.ds(last * W, W)],
                        rows_v.at[pl.ds(CH * W, W)])

        def row_body(y2, _):
            for g in range(W // 16):
                c = g * 16 + lanes
                k = c >> 2
                m0 = c & 3
                r_hi = y2 * K + k
                r_lo = y2 * K + jnp.maximum(k - 1, 0)
                msk = c >= 4
                vy = rows_v[pl.ds(y2 * W + g * 16, 16)]
                vy1 = rows_v[pl.ds((y2 + 1) * W + g * 16, 16)]
                plsc.store_scatter(trow_v, [r_hi, m0], vy)
                plsc.store_scatter(trow_v, [r_lo, m0 + 4], vy, mask=msk)
                plsc.store_scatter(trow_v, [r_hi, m0 + 8], vy1)
                plsc.store_scatter(trow_v, [r_lo, m0 + 12], vy1, mask=msk)
            return 0

        lax.fori_loop(0, CH, row_body, 0)
        pltpu.sync_copy(trow_v, tab.at[pl.ds(gy * K, CH * K)])
        return 0

    lax.fori_loop(0, ROWS_PER_W // CH, chunk, 0)
    plsc.subcore_barrier()

    row0 = (wid % 2) * ROWS_PER_W + b * H

    bufs = ((idx0_v, off0_v, gth0_v, wgt0_v), (idx1_v, off1_v, gth1_v, wgt1_v))
    wrows = (wrow0_v, wrow1_v)

    def prefetch(rg, par):
        rgc = jnp.minimum(rg, B * H - 1)
        pltpu.async_copy(wxy.at[pl.ds(rgc * 2 * W, 2 * W)], wrows[par],
                         wsem.at[par])

    def gen(yrow, par):
        idx_v, off_v, _, wgt_v = bufs[par]
        wrow_v = wrows[par]
        pltpu.make_async_copy(wxy.at[pl.ds(0, 2 * W)], wrow_v,
                              wsem.at[par]).wait()
        yrow_f = yrow.astype(jnp.float32)
        for g in range(W // 16):
            j = g * 16 + lanes
            wx = wrow_v[pl.ds(g * 16, 16)]
            wy = wrow_v[pl.ds(W + g * 16, 16)]
            fx = j.astype(jnp.float32) + wx
            fy = yrow_f + wy
            cx = jnp.clip(fx.astype(jnp.int32), 0, W - 2)
            cy = jnp.clip(fy.astype(jnp.int32), 0, H - 2)
            dx = fx - cx.astype(jnp.float32)
            dy = fy - cy.astype(jnp.float32)
            o = g * 16
            idx_v[pl.ds(o, 16)] = (b * H + cy) * K + (cx >> 2)
            off_v[pl.ds(o, 16)] = cx & 3
            wgt_v[pl.ds(o, 16)] = dx
            wgt_v[pl.ds(W + o, 16)] = dy

    def fire(par):
        idx_v, _, gth_v, _ = bufs[par]
        return pltpu.async_copy(tab.at[idx_v], gth_v, gsem.at[par])

    def comb_row(rg, par):
        idx_v, off_v, gth_v, wgt_v = bufs[par]
        pltpu.make_async_copy(tab.at[idx_v], gth_v, gsem.at[par]).wait()
        for g in range(W // 16):
            o = g * 16
            prow = o + lanes
            off = off_v[pl.ds(o, 16)]
            g00 = plsc.load_gather(gth_v, [prow, off])
            g01 = plsc.load_gather(gth_v, [prow, off + 1])
            g10 = plsc.load_gather(gth_v, [prow, off + 8])
            g11 = plsc.load_gather(gth_v, [prow, off + 9])
            dx = wgt_v[pl.ds(o, 16)]
            dy = wgt_v[pl.ds(W + o, 16)]
            ndx = 1.0 - dx
            ndy = 1.0 - dy
            out_v[pl.ds(o, 16)] = (g00 * ndx * ndy + g01 * dx * ndy
                                   + g11 * dx * dy + g10 * ndx * dy)
        pltpu.sync_copy(out_v, out.at[pl.ds(rg * W, W)])

    prefetch(row0, 0)
    prefetch(row0 + 1, 1)
    gen(y0, 0)
    fire(0)

    def do_pair(rr, _):
        r1 = 2 * rr + 1
        prefetch(row0 + r1 + 1, 0)
        gen(y0 + r1, 1)
        fire(1)
        comb_row(row0 + r1 - 1, 0)
        r2 = r1 + 1
        prefetch(row0 + r2 + 1, 1)
        gen(y0 + r2, 0)
        fire(0)
        comb_row(row0 + r2 - 1, 1)
        return 0

    lax.fori_loop(0, (ROWS_PER_W - 2) // 2, do_pair, 0)
    rl = ROWS_PER_W - 1
    gen(y0 + rl, 1)
    fire(1)
    comb_row(row0 + rl - 1, 0)
    comb_row(row0 + rl, 1)


@jax.jit
def kernel(x):
    xt = jnp.transpose(x, (0, 3, 1, 2))
    plane = jax.ShapeDtypeStruct((B, H * W // 128, 128), jnp.float32)
    wplane = jax.ShapeDtypeStruct((B, 2 * H * W // 128, 128), jnp.float32)
    inp3, wxy3 = pl.pallas_call(
        _reformat_body,
        grid=(B,),
        in_specs=[pl.BlockSpec((1, 3, H, W), lambda b: (b, 0, 0, 0))],
        out_specs=[
            pl.BlockSpec((1, H * W // 128, 128), lambda b: (b, 0, 0)),
            pl.BlockSpec((1, 2 * H * W // 128, 128), lambda b: (b, 0, 0)),
        ],
        out_shape=[plane, wplane],
    )(xt)
    inp1 = inp3.reshape(NPIX)
    wxy1 = wxy3.reshape(2 * NPIX)

    mesh = plsc.VectorSubcoreMesh(core_axis_name="c", subcore_axis_name="s")
    sc_params = pltpu.CompilerParams(
        needs_layout_passes=False, use_tc_tiling_on_sc=False)

    fused = pl.kernel(
        _fused_body,
        out_type=[
            jax.ShapeDtypeStruct((NPIX,), jnp.float32),
            jax.ShapeDtypeStruct((B * H * K, 16), jnp.float32),
        ],
        mesh=mesh,
        scratch_types=[
            pltpu.VMEM(((CH + 1) * W,), jnp.float32),
            pltpu.VMEM((CH * K, 16), jnp.float32),
            pltpu.VMEM((2 * W,), jnp.float32),
            pltpu.VMEM((2 * W,), jnp.float32),
            pltpu.VMEM((W,), jnp.int32),
            pltpu.VMEM((W,), jnp.int32),
            pltpu.VMEM((W,), jnp.int32),
            pltpu.VMEM((W,), jnp.int32),
            pltpu.VMEM((W, 16), jnp.float32),
            pltpu.VMEM((W, 16), jnp.float32),
            pltpu.VMEM((2 * W,), jnp.float32),
            pltpu.VMEM((2 * W,), jnp.float32),
            pltpu.VMEM((W,), jnp.float32),
            pltpu.SemaphoreType.DMA((2,)),
            pltpu.SemaphoreType.DMA((2,)),
        ],
        compiler_params=sc_params,
    )
    y, _ = fused(inp1, wxy1)
    return y.reshape(B, H, W, 1)
